# baseline (speedup 1.0000x reference)
import functools

import jax
import jax.numpy as jnp
from jax import lax
from jax.experimental import pallas as pl
from jax.experimental.pallas import tpu as pltpu
from jax.experimental.pallas import tpu_sc as plsc

USER = 10000
ITEM = 6000
ENTITY = 10000
LATDIM = 128
NREL = 16
N_HOPS = 2
GNN_LAYER = 2
RES_LAMBDA = 0.5
NNZ_ADJ = 512000
E_KG = 320000
N_GRAPH = USER + ITEM

NC = 2
NS = 16
NW = NC * NS
L = 16

CH = 128
EPT_KG = 10112
E_KG_PAD = EPT_KG * NW
DUMP_E = ENTITY
HALF = N_GRAPH // 2
DUMP = HALF

_SC_MESH = plsc.VectorSubcoreMesh(
    core_axis_name="c", subcore_axis_name="s", num_cores=NC, num_subcores=NS)


def _rgat_body(emb, ahf, atf, idx4, zmat, zcol,
               agg_out, s_out,
               acc, sacc, idxb, hds, ahv2, atv2, exv2, rows2, sv,
               isem, gsem, ssem):
    cid = lax.axis_index("c")
    sid = lax.axis_index("s")
    wid = sid * NC + cid
    nch = EPT_KG // CH
    rpt = 1000
    c0 = wid * nch

    @pl.when(sid < ENTITY // rpt)
    def _():
        pltpu.sync_copy(zmat.at[pl.ds(sid * rpt, rpt), :],
                        acc.at[pl.ds(sid * rpt, rpt), :])
        pltpu.sync_copy(zcol.at[pl.ds(sid * rpt, rpt)], sv)
        pltpu.sync_copy(sv, sacc.at[pl.ds(sid * rpt, rpt)])

    plsc.subcore_barrier()

    def fire_idx(j, p):
        pltpu.async_copy(idx4.at[pl.ds((c0 + j) * (4 * CH), 4 * CH)],
                         idxb.at[p], isem.at[p])

    def wait_idx(j, p):
        pltpu.make_async_copy(idx4.at[pl.ds((c0 + j) * (4 * CH), 4 * CH)],
                              idxb.at[p], isem.at[p]).wait()

    def fire_data(p):
        pltpu.async_copy(ahf.at[idxb.at[p, pl.ds(0, CH)]],
                         ahv2.at[p], gsem.at[p])
        pltpu.async_copy(atf.at[idxb.at[p, pl.ds(CH, CH)]],
                         atv2.at[p], gsem.at[p])
        pltpu.async_copy(emb.at[idxb.at[p, pl.ds(2 * CH, CH)]],
                         rows2.at[p], gsem.at[p])

    def wait_data(p):
        pltpu.make_async_copy(ahf.at[idxb.at[p, pl.ds(0, CH)]],
                              ahv2.at[p], gsem.at[p]).wait()
        pltpu.make_async_copy(atf.at[idxb.at[p, pl.ds(CH, CH)]],
                              atv2.at[p], gsem.at[p]).wait()
        pltpu.make_async_copy(emb.at[idxb.at[p, pl.ds(2 * CH, CH)]],
                              rows2.at[p], gsem.at[p]).wait()

    def fire_scatter(p):
        pltpu.async_copy(exv2.at[p], sacc.at[hds.at[p]], ssem.at[p],
                         add=True)
        pltpu.async_copy(rows2.at[p], acc.at[hds.at[p]], ssem.at[p],
                         add=True)

    def drain_scatter(p):
        pltpu.make_async_copy(exv2.at[p], sacc.at[hds.at[p]],
                              ssem.at[p]).wait()
        pltpu.make_async_copy(rows2.at[p], acc.at[hds.at[p]],
                              ssem.at[p]).wait()

    fire_idx(0, 0)
    fire_idx(1, 1)
    wait_idx(0, 0)
    fire_data(0)

    def body(j, _):
        p = lax.rem(j, 2)
        pn = lax.rem(j + 1, 2)
        wait_data(p)

        @pl.when(j >= 1)
        def _():
            drain_scatter(pn)

        @pl.when(j + 1 < nch)
        def _():
            wait_idx(j + 1, pn)
            fire_data(pn)

        for g in range(CH // L):
            e = ahv2[p, pl.ds(g * L, L)] + atv2[p, pl.ds(g * L, L)]
            e = jnp.where(e >= 0.0, e, 0.2 * e)
            exv2[p, pl.ds(g * L, L)] = jnp.exp(e)
            hds[p, pl.ds(g * L, L)] = idxb[p, pl.ds(3 * CH + g * L, L)]

        for g in range(CH // L):
            scv = exv2[p, pl.ds(g * L, L)]
            for k in range(L):
                sc = scv[k]
                i = g * L + k
                for u in range(LATDIM // L):
                    rows2[p, i, pl.ds(u * L, L)] = (
                        rows2[p, i, pl.ds(u * L, L)] * sc)

        fire_scatter(p)

        @pl.when(j + 2 < nch)
        def _():
            fire_idx(j + 2, p)

        return 0

    lax.fori_loop(0, nch, body, 0)
    drain_scatter(lax.rem(nch - 1, 2))
    plsc.subcore_barrier()

    @pl.when(sid < ENTITY // rpt)
    def _():
        pltpu.sync_copy(acc.at[pl.ds(sid * rpt, rpt), :],
                        agg_out.at[cid, pl.ds(sid * rpt, rpt), :])
        pltpu.sync_copy(sacc.at[pl.ds(sid * rpt, rpt)], sv)
        pltpu.sync_copy(sv, s_out.at[pl.ds(cid * ENTITY + sid * rpt, rpt)])


def _rgat(emb, ahf, atf, idx4, zmat, zcol):
    f = pl.kernel(
        _rgat_body,
        out_type=(jax.ShapeDtypeStruct((NC, ENTITY, LATDIM), jnp.float32),
                  jax.ShapeDtypeStruct((NC * ENTITY,), jnp.float32)),
        mesh=_SC_MESH,
        scratch_types=[
            pltpu.VMEM_SHARED((ENTITY + 8, LATDIM), jnp.float32),
            pltpu.VMEM_SHARED((ENTITY + 8,), jnp.float32),
            pltpu.VMEM((2, 4 * CH), jnp.int32),
            pltpu.VMEM((2, CH), jnp.int32),
            pltpu.VMEM((2, CH), jnp.float32),
            pltpu.VMEM((2, CH), jnp.float32),
            pltpu.VMEM((2, CH), jnp.float32),
            pltpu.VMEM((2, CH, LATDIM), jnp.float32),
            pltpu.VMEM((1000,), jnp.float32),
            pltpu.SemaphoreType.DMA((2,)),
            pltpu.SemaphoreType.DMA((2,)),
            pltpu.SemaphoreType.DMA((2,)),
        ],
    )
    return f(emb, ahf, atf, idx4, zmat, zcol)


def _spmm_body(prev, idx3, zmat,
               part_out,
               acc, idxb, rowm, rows2, isem, gsem, ssem):
    cid = lax.axis_index("c")
    sid = lax.axis_index("s")
    ept = NNZ_ADJ // NS
    nch = ept // CH
    rpt = 1000
    c0 = sid * nch
    lo = cid * HALF

    @pl.when(sid < HALF // rpt)
    def _():
        pltpu.sync_copy(zmat.at[pl.ds(sid * rpt, rpt), :],
                        acc.at[pl.ds(sid * rpt, rpt), :])

    plsc.subcore_barrier()

    def fire_idx(j, p):
        pltpu.async_copy(idx3.at[pl.ds((c0 + j) * (3 * CH), 3 * CH)],
                         idxb.at[p], isem.at[p])

    def wait_idx(j, p):
        pltpu.make_async_copy(idx3.at[pl.ds((c0 + j) * (3 * CH), 3 * CH)],
                              idxb.at[p], isem.at[p]).wait()

    def fire_data(p):
        pltpu.async_copy(prev.at[idxb.at[p, pl.ds(0, CH)]],
                         rows2.at[p], gsem.at[p])

    def wait_data(p):
        pltpu.make_async_copy(prev.at[idxb.at[p, pl.ds(0, CH)]],
                              rows2.at[p], gsem.at[p]).wait()

    def fire_scatter(p):
        pltpu.async_copy(rows2.at[p], acc.at[rowm.at[p]], ssem.at[p],
                         add=True)

    def drain_scatter(p):
        pltpu.make_async_copy(rows2.at[p], acc.at[rowm.at[p]],
                              ssem.at[p]).wait()

    fire_idx(0, 0)
    fire_idx(1, 1)
    wait_idx(0, 0)
    fire_data(0)

    def body(j, _):
        p = lax.rem(j, 2)
        pn = lax.rem(j + 1, 2)
        wait_data(p)

        @pl.when(j >= 1)
        def _():
            drain_scatter(pn)

        @pl.when(j + 1 < nch)
        def _():
            wait_idx(j + 1, pn)
            fire_data(pn)

        for g in range(CH // L):
            r = idxb[p, pl.ds(CH + g * L, L)]
            rl = r - lo
            inr = (rl >= 0) & (rl < HALF)
            rowm[p, pl.ds(g * L, L)] = jnp.where(inr, rl, DUMP)

        for g in range(CH // L):
            scv = lax.bitcast_convert_type(
                idxb[p, pl.ds(2 * CH + g * L, L)], jnp.float32)
            for k in range(L):
                sc = scv[k]
                i = g * L + k
                for u in range(LATDIM // L):
                    rows2[p, i, pl.ds(u * L, L)] = (
                        rows2[p, i, pl.ds(u * L, L)] * sc)

        fire_scatter(p)

        @pl.when(j + 2 < nch)
        def _():
            fire_idx(j + 2, p)

        return 0

    lax.fori_loop(0, nch, body, 0)
    drain_scatter(lax.rem(nch - 1, 2))
    plsc.subcore_barrier()

    @pl.when(sid < HALF // rpt)
    def _():
        pltpu.sync_copy(acc.at[pl.ds(sid * rpt, rpt), :],
                        part_out.at[cid, pl.ds(sid * rpt, rpt), :])


def _spmm(prev, idx3, zmat):
    f = pl.kernel(
        _spmm_body,
        out_type=jax.ShapeDtypeStruct((NC, HALF, LATDIM), jnp.float32),
        mesh=_SC_MESH,
        scratch_types=[
            pltpu.VMEM_SHARED((HALF + 8, LATDIM), jnp.float32),
            pltpu.VMEM((2, 3 * CH), jnp.int32),
            pltpu.VMEM((2, CH), jnp.int32),
            pltpu.VMEM((2, CH, LATDIM), jnp.float32),
            pltpu.SemaphoreType.DMA((2,)),
            pltpu.SemaphoreType.DMA((2,)),
            pltpu.SemaphoreType.DMA((2,)),
        ],
    )
    return f(prev, idx3, zmat)


def _score_tables(e, w_ref, r_ref):
    wr1 = lax.dot_general(w_ref[:LATDIM, :], r_ref[...],
                          (((1,), (1,)), ((), ())),
                          preferred_element_type=jnp.float32)
    wr2 = lax.dot_general(w_ref[LATDIM:, :], r_ref[...],
                          (((1,), (1,)), ((), ())),
                          preferred_element_type=jnp.float32)
    ah = jnp.dot(e, wr1, preferred_element_type=jnp.float32)
    at = jnp.dot(e, wr2, preferred_element_type=jnp.float32)
    return ah, at


def _prep_body(emb_ref, w_ref, r_ref, ah_ref, at_ref):
    ah, at = _score_tables(emb_ref[...], w_ref, r_ref)
    ah_ref[...] = ah
    at_ref[...] = at


def _prep(emb, w, r):
    nb = 10
    br = ENTITY // nb
    return pl.pallas_call(
        _prep_body,
        grid=(nb,),
        in_specs=[
            pl.BlockSpec((br, LATDIM), lambda i: (i, 0)),
            pl.BlockSpec((2 * LATDIM, LATDIM), lambda i: (0, 0)),
            pl.BlockSpec((NREL, LATDIM), lambda i: (0, 0)),
        ],
        out_specs=[
            pl.BlockSpec((br, NREL), lambda i: (i, 0)),
            pl.BlockSpec((br, NREL), lambda i: (i, 0)),
        ],
        out_shape=[
            jax.ShapeDtypeStruct((ENTITY, NREL), jnp.float32),
            jax.ShapeDtypeStruct((ENTITY, NREL), jnp.float32),
        ],
    )(emb, w, r)


def _hop_body(aggp_ref, sp_ref, emb_ref, res_ref, w_ref, r_ref,
              embo_ref, reso_ref, ah_ref, at_ref):
    agg = aggp_ref[0] + aggp_ref[1]
    s2 = sp_ref[...]
    s = s2[:, 0:1] + s2[:, 1:2]
    denom = jnp.where(s == 0.0, 1.0, s)
    x = agg / denom + emb_ref[...]
    n = jnp.sqrt(jnp.sum(x * x, axis=-1, keepdims=True))
    e = x / jnp.maximum(n, 1e-12)
    embo_ref[...] = e
    reso_ref[...] = RES_LAMBDA * res_ref[...] + e
    ah, at = _score_tables(e, w_ref, r_ref)
    ah_ref[...] = ah
    at_ref[...] = at


def _hop_update(aggp, sp, emb, res, w, r):
    nb = 10
    br = ENTITY // nb
    return pl.pallas_call(
        _hop_body,
        grid=(nb,),
        in_specs=[
            pl.BlockSpec((NC, br, LATDIM), lambda i: (0, i, 0)),
            pl.BlockSpec((br, NC), lambda i: (i, 0)),
            pl.BlockSpec((br, LATDIM), lambda i: (i, 0)),
            pl.BlockSpec((br, LATDIM), lambda i: (i, 0)),
            pl.BlockSpec((2 * LATDIM, LATDIM), lambda i: (0, 0)),
            pl.BlockSpec((NREL, LATDIM), lambda i: (0, 0)),
        ],
        out_specs=[
            pl.BlockSpec((br, LATDIM), lambda i: (i, 0)),
            pl.BlockSpec((br, LATDIM), lambda i: (i, 0)),
            pl.BlockSpec((br, NREL), lambda i: (i, 0)),
            pl.BlockSpec((br, NREL), lambda i: (i, 0)),
        ],
        out_shape=[
            jax.ShapeDtypeStruct((ENTITY, LATDIM), jnp.float32),
            jax.ShapeDtypeStruct((ENTITY, LATDIM), jnp.float32),
            jax.ShapeDtypeStruct((ENTITY, NREL), jnp.float32),
            jax.ShapeDtypeStruct((ENTITY, NREL), jnp.float32),
        ],
    )(aggp, sp, emb, res, w, r)


def _comb_body(e0_ref, g1_ref, g2_ref, o_ref):
    o_ref[...] = e0_ref[...] + g1_ref[...] + g2_ref[...]


def _combine(e0, g1, g2):
    nb = 16
    br = N_GRAPH // nb
    return pl.pallas_call(
        _comb_body,
        grid=(nb,),
        in_specs=[
            pl.BlockSpec((br, LATDIM), lambda i: (i, 0)),
            pl.BlockSpec((br, LATDIM), lambda i: (i, 0)),
            pl.BlockSpec((br, LATDIM), lambda i: (i, 0)),
        ],
        out_specs=pl.BlockSpec((br, LATDIM), lambda i: (i, 0)),
        out_shape=jax.ShapeDtypeStruct((N_GRAPH, LATDIM), jnp.float32),
    )(e0, g1, g2)


def kernel(adj_rows, adj_cols, adj_vals, edge_index, edge_type,
           uEmbeds, eEmbeds, rEmbeds, W):
    head = edge_index[0].astype(jnp.int32)
    tail = edge_index[1].astype(jnp.int32)
    et = edge_type.astype(jnp.int32)

    npad = E_KG_PAD - E_KG
    ih = jnp.pad(head * NREL + et, (0, npad))
    it = jnp.pad(tail * NREL + et, (0, npad))
    tl = jnp.pad(tail, (0, npad))
    hd = jnp.pad(head, (0, npad), constant_values=DUMP_E)
    idx4 = jnp.stack([ih.reshape(-1, CH), it.reshape(-1, CH),
                      tl.reshape(-1, CH), hd.reshape(-1, CH)],
                     axis=1).reshape(-1)

    idx3 = jnp.stack(
        [adj_cols.astype(jnp.int32).reshape(-1, CH),
         adj_rows.astype(jnp.int32).reshape(-1, CH),
         lax.bitcast_convert_type(adj_vals, jnp.int32).reshape(-1, CH)],
        axis=1).reshape(-1)

    zmat_e = jnp.zeros((ENTITY, LATDIM), jnp.float32)
    zcol_e = jnp.zeros((ENTITY,), jnp.float32)

    ah, at = _prep(eEmbeds, W, rEmbeds)
    emb = eEmbeds
    res = eEmbeds
    for _ in range(N_HOPS):
        aggp, sp = _rgat(emb, ah.reshape(-1), at.reshape(-1),
                         idx4, zmat_e, zcol_e)
        emb, res, ah, at = _hop_update(aggp, sp.reshape(NC, ENTITY).T,
                                       emb, res, W, rEmbeds)

    embeds0 = jnp.concatenate([uEmbeds, res[:ITEM]], axis=0)
    g1 = _spmm(embeds0, idx3, zmat_e).reshape(N_GRAPH, LATDIM)
    g2 = _spmm(g1, idx3, zmat_e).reshape(N_GRAPH, LATDIM)
    total = _combine(embeds0, g1, g2)
    return total[:USER], total[USER:]


# speedup vs baseline: 1.0539x; 1.0539x over previous
import functools

import jax
import jax.numpy as jnp
from jax import lax
from jax.experimental import pallas as pl
from jax.experimental.pallas import tpu as pltpu
from jax.experimental.pallas import tpu_sc as plsc

USER = 10000
ITEM = 6000
ENTITY = 10000
LATDIM = 128
NREL = 16
N_HOPS = 2
GNN_LAYER = 2
RES_LAMBDA = 0.5
NNZ_ADJ = 512000
E_KG = 320000
N_GRAPH = USER + ITEM

NC = 2
NS = 16
NW = NC * NS
L = 16

CH = 128
EPT_KG = 10112
E_KG_PAD = EPT_KG * NW
DUMP_E = ENTITY
HALF = N_GRAPH // 2
DUMP = HALF

_SC_MESH = plsc.VectorSubcoreMesh(
    core_axis_name="c", subcore_axis_name="s", num_cores=NC, num_subcores=NS)


def _rgat_body(emb, ahf, atf, ih, it, tl, hd, zmat, zcol,
               agg_out, s_out,
               acc, sacc, ihb, itb, tlb, hdb, hds, ahv2, atv2, exv2, rows2,
               sv, isem, gsem, ssem):
    cid = lax.axis_index("c")
    sid = lax.axis_index("s")
    wid = sid * NC + cid
    nch = EPT_KG // CH
    rpt = 1000
    c0 = wid * nch

    @pl.when(sid < ENTITY // rpt)
    def _():
        pltpu.sync_copy(zmat.at[pl.ds(sid * rpt, rpt), :],
                        acc.at[pl.ds(sid * rpt, rpt), :])
        pltpu.sync_copy(zcol.at[pl.ds(sid * rpt, rpt)], sv)
        pltpu.sync_copy(sv, sacc.at[pl.ds(sid * rpt, rpt)])

    plsc.subcore_barrier()

    def fire_idx(j, p):
        b = (c0 + j) * CH
        pltpu.async_copy(ih.at[pl.ds(b, CH)], ihb.at[p], isem.at[p])
        pltpu.async_copy(it.at[pl.ds(b, CH)], itb.at[p], isem.at[p])
        pltpu.async_copy(tl.at[pl.ds(b, CH)], tlb.at[p], isem.at[p])
        pltpu.async_copy(hd.at[pl.ds(b, CH)], hdb.at[p], isem.at[p])

    def wait_idx(j, p):
        b = (c0 + j) * CH
        pltpu.make_async_copy(ih.at[pl.ds(b, CH)], ihb.at[p],
                              isem.at[p]).wait()
        pltpu.make_async_copy(it.at[pl.ds(b, CH)], itb.at[p],
                              isem.at[p]).wait()
        pltpu.make_async_copy(tl.at[pl.ds(b, CH)], tlb.at[p],
                              isem.at[p]).wait()
        pltpu.make_async_copy(hd.at[pl.ds(b, CH)], hdb.at[p],
                              isem.at[p]).wait()

    def fire_data(p):
        pltpu.async_copy(ahf.at[ihb.at[p]], ahv2.at[p], gsem.at[p])
        pltpu.async_copy(atf.at[itb.at[p]], atv2.at[p], gsem.at[p])
        pltpu.async_copy(emb.at[tlb.at[p]], rows2.at[p], gsem.at[p])

    def wait_data(p):
        pltpu.make_async_copy(ahf.at[ihb.at[p]], ahv2.at[p],
                              gsem.at[p]).wait()
        pltpu.make_async_copy(atf.at[itb.at[p]], atv2.at[p],
                              gsem.at[p]).wait()
        pltpu.make_async_copy(emb.at[tlb.at[p]], rows2.at[p],
                              gsem.at[p]).wait()

    def fire_scatter(p):
        pltpu.async_copy(exv2.at[p], sacc.at[hds.at[p]], ssem.at[p],
                         add=True)
        pltpu.async_copy(rows2.at[p], acc.at[hds.at[p]], ssem.at[p],
                         add=True)

    def drain_scatter(p):
        pltpu.make_async_copy(exv2.at[p], sacc.at[hds.at[p]],
                              ssem.at[p]).wait()
        pltpu.make_async_copy(rows2.at[p], acc.at[hds.at[p]],
                              ssem.at[p]).wait()

    fire_idx(0, 0)
    fire_idx(1, 1)
    wait_idx(0, 0)
    fire_data(0)

    def body(j, _):
        p = lax.rem(j, 2)
        pn = lax.rem(j + 1, 2)
        wait_data(p)

        @pl.when(j >= 1)
        def _():
            drain_scatter(pn)

        @pl.when(j + 1 < nch)
        def _():
            wait_idx(j + 1, pn)
            fire_data(pn)

        for g in range(CH // L):
            e = ahv2[p, pl.ds(g * L, L)] + atv2[p, pl.ds(g * L, L)]
            e = jnp.where(e >= 0.0, e, 0.2 * e)
            exv2[p, pl.ds(g * L, L)] = jnp.exp(e)
            hds[p, pl.ds(g * L, L)] = hdb[p, pl.ds(g * L, L)]

        for g in range(CH // L):
            scv = exv2[p, pl.ds(g * L, L)]
            for k in range(L):
                sc = scv[k]
                i = g * L + k
                for u in range(LATDIM // L):
                    rows2[p, i, pl.ds(u * L, L)] = (
                        rows2[p, i, pl.ds(u * L, L)] * sc)

        fire_scatter(p)

        @pl.when(j + 2 < nch)
        def _():
            fire_idx(j + 2, p)

        return 0

    lax.fori_loop(0, nch, body, 0)
    drain_scatter(lax.rem(nch - 1, 2))
    plsc.subcore_barrier()

    @pl.when(sid < ENTITY // rpt)
    def _():
        pltpu.sync_copy(acc.at[pl.ds(sid * rpt, rpt), :],
                        agg_out.at[cid, pl.ds(sid * rpt, rpt), :])
        pltpu.sync_copy(sacc.at[pl.ds(sid * rpt, rpt)], sv)
        pltpu.sync_copy(sv, s_out.at[pl.ds(cid * ENTITY + sid * rpt, rpt)])


def _rgat(emb, ahf, atf, ih, it, tl, hd, zmat, zcol):
    f = pl.kernel(
        _rgat_body,
        out_type=(jax.ShapeDtypeStruct((NC, ENTITY, LATDIM), jnp.float32),
                  jax.ShapeDtypeStruct((NC * ENTITY,), jnp.float32)),
        mesh=_SC_MESH,
        scratch_types=[
            pltpu.VMEM_SHARED((ENTITY + 8, LATDIM), jnp.float32),
            pltpu.VMEM_SHARED((ENTITY + 8,), jnp.float32),
            pltpu.VMEM((2, CH), jnp.int32),
            pltpu.VMEM((2, CH), jnp.int32),
            pltpu.VMEM((2, CH), jnp.int32),
            pltpu.VMEM((2, CH), jnp.int32),
            pltpu.VMEM((2, CH), jnp.int32),
            pltpu.VMEM((2, CH), jnp.float32),
            pltpu.VMEM((2, CH), jnp.float32),
            pltpu.VMEM((2, CH), jnp.float32),
            pltpu.VMEM((2, CH, LATDIM), jnp.float32),
            pltpu.VMEM((1000,), jnp.float32),
            pltpu.SemaphoreType.DMA((2,)),
            pltpu.SemaphoreType.DMA((2,)),
            pltpu.SemaphoreType.DMA((2,)),
        ],
    )
    return f(emb, ahf, atf, ih, it, tl, hd, zmat, zcol)


def _spmm_body(prev, cols, rows, vals, zmat,
               part_out,
               acc, colb, rowb, valb, rowm, rows2, isem, gsem, ssem):
    cid = lax.axis_index("c")
    sid = lax.axis_index("s")
    ept = NNZ_ADJ // NS
    nch = ept // CH
    rpt = 1000
    c0 = sid * nch
    lo = cid * HALF

    @pl.when(sid < HALF // rpt)
    def _():
        pltpu.sync_copy(zmat.at[pl.ds(sid * rpt, rpt), :],
                        acc.at[pl.ds(sid * rpt, rpt), :])

    plsc.subcore_barrier()

    def fire_idx(j, p):
        b = (c0 + j) * CH
        pltpu.async_copy(cols.at[pl.ds(b, CH)], colb.at[p], isem.at[p])
        pltpu.async_copy(rows.at[pl.ds(b, CH)], rowb.at[p], isem.at[p])
        pltpu.async_copy(vals.at[pl.ds(b, CH)], valb.at[p], isem.at[p])

    def wait_idx(j, p):
        b = (c0 + j) * CH
        pltpu.make_async_copy(cols.at[pl.ds(b, CH)], colb.at[p],
                              isem.at[p]).wait()
        pltpu.make_async_copy(rows.at[pl.ds(b, CH)], rowb.at[p],
                              isem.at[p]).wait()
        pltpu.make_async_copy(vals.at[pl.ds(b, CH)], valb.at[p],
                              isem.at[p]).wait()

    def fire_data(p):
        pltpu.async_copy(prev.at[colb.at[p]], rows2.at[p], gsem.at[p])

    def wait_data(p):
        pltpu.make_async_copy(prev.at[colb.at[p]], rows2.at[p],
                              gsem.at[p]).wait()

    def fire_scatter(p):
        pltpu.async_copy(rows2.at[p], acc.at[rowm.at[p]], ssem.at[p],
                         add=True)

    def drain_scatter(p):
        pltpu.make_async_copy(rows2.at[p], acc.at[rowm.at[p]],
                              ssem.at[p]).wait()

    fire_idx(0, 0)
    fire_idx(1, 1)
    wait_idx(0, 0)
    fire_data(0)

    def body(j, _):
        p = lax.rem(j, 2)
        pn = lax.rem(j + 1, 2)
        wait_data(p)

        @pl.when(j >= 1)
        def _():
            drain_scatter(pn)

        @pl.when(j + 1 < nch)
        def _():
            wait_idx(j + 1, pn)
            fire_data(pn)

        for g in range(CH // L):
            r = rowb[p, pl.ds(g * L, L)]
            rl = r - lo
            inr = (rl >= 0) & (rl < HALF)
            rowm[p, pl.ds(g * L, L)] = jnp.where(inr, rl, DUMP)

        for g in range(CH // L):
            scv = valb[p, pl.ds(g * L, L)]
            for k in range(L):
                sc = scv[k]
                i = g * L + k
                for u in range(LATDIM // L):
                    rows2[p, i, pl.ds(u * L, L)] = (
                        rows2[p, i, pl.ds(u * L, L)] * sc)

        fire_scatter(p)

        @pl.when(j + 2 < nch)
        def _():
            fire_idx(j + 2, p)

        return 0

    lax.fori_loop(0, nch, body, 0)
    drain_scatter(lax.rem(nch - 1, 2))
    plsc.subcore_barrier()

    @pl.when(sid < HALF // rpt)
    def _():
        pltpu.sync_copy(acc.at[pl.ds(sid * rpt, rpt), :],
                        part_out.at[cid, pl.ds(sid * rpt, rpt), :])


def _spmm(prev, cols, rows, vals, zmat):
    f = pl.kernel(
        _spmm_body,
        out_type=jax.ShapeDtypeStruct((NC, HALF, LATDIM), jnp.float32),
        mesh=_SC_MESH,
        scratch_types=[
            pltpu.VMEM_SHARED((HALF + 8, LATDIM), jnp.float32),
            pltpu.VMEM((2, CH), jnp.int32),
            pltpu.VMEM((2, CH), jnp.int32),
            pltpu.VMEM((2, CH), jnp.float32),
            pltpu.VMEM((2, CH), jnp.int32),
            pltpu.VMEM((2, CH, LATDIM), jnp.float32),
            pltpu.SemaphoreType.DMA((2,)),
            pltpu.SemaphoreType.DMA((2,)),
            pltpu.SemaphoreType.DMA((2,)),
        ],
    )
    return f(prev, cols, rows, vals, zmat)


def _score_tables(e, w_ref, r_ref):
    wr1 = lax.dot_general(w_ref[:LATDIM, :], r_ref[...],
                          (((1,), (1,)), ((), ())),
                          preferred_element_type=jnp.float32)
    wr2 = lax.dot_general(w_ref[LATDIM:, :], r_ref[...],
                          (((1,), (1,)), ((), ())),
                          preferred_element_type=jnp.float32)
    ah = jnp.dot(e, wr1, preferred_element_type=jnp.float32)
    at = jnp.dot(e, wr2, preferred_element_type=jnp.float32)
    return ah, at


def _prep_body(emb_ref, w_ref, r_ref, ah_ref, at_ref):
    ah, at = _score_tables(emb_ref[...], w_ref, r_ref)
    ah_ref[...] = ah
    at_ref[...] = at


def _prep(emb, w, r):
    nb = 10
    br = ENTITY // nb
    return pl.pallas_call(
        _prep_body,
        grid=(nb,),
        in_specs=[
            pl.BlockSpec((br, LATDIM), lambda i: (i, 0)),
            pl.BlockSpec((2 * LATDIM, LATDIM), lambda i: (0, 0)),
            pl.BlockSpec((NREL, LATDIM), lambda i: (0, 0)),
        ],
        out_specs=[
            pl.BlockSpec((br, NREL), lambda i: (i, 0)),
            pl.BlockSpec((br, NREL), lambda i: (i, 0)),
        ],
        out_shape=[
            jax.ShapeDtypeStruct((ENTITY, NREL), jnp.float32),
            jax.ShapeDtypeStruct((ENTITY, NREL), jnp.float32),
        ],
    )(emb, w, r)


def _hop_body(aggp_ref, sp_ref, emb_ref, res_ref, w_ref, r_ref,
              embo_ref, reso_ref, ah_ref, at_ref):
    agg = aggp_ref[0] + aggp_ref[1]
    s2 = sp_ref[...]
    s = s2[:, 0:1] + s2[:, 1:2]
    denom = jnp.where(s == 0.0, 1.0, s)
    x = agg / denom + emb_ref[...]
    n = jnp.sqrt(jnp.sum(x * x, axis=-1, keepdims=True))
    e = x / jnp.maximum(n, 1e-12)
    embo_ref[...] = e
    reso_ref[...] = RES_LAMBDA * res_ref[...] + e
    ah, at = _score_tables(e, w_ref, r_ref)
    ah_ref[...] = ah
    at_ref[...] = at


def _hop_update(aggp, sp, emb, res, w, r):
    nb = 10
    br = ENTITY // nb
    return pl.pallas_call(
        _hop_body,
        grid=(nb,),
        in_specs=[
            pl.BlockSpec((NC, br, LATDIM), lambda i: (0, i, 0)),
            pl.BlockSpec((br, NC), lambda i: (i, 0)),
            pl.BlockSpec((br, LATDIM), lambda i: (i, 0)),
            pl.BlockSpec((br, LATDIM), lambda i: (i, 0)),
            pl.BlockSpec((2 * LATDIM, LATDIM), lambda i: (0, 0)),
            pl.BlockSpec((NREL, LATDIM), lambda i: (0, 0)),
        ],
        out_specs=[
            pl.BlockSpec((br, LATDIM), lambda i: (i, 0)),
            pl.BlockSpec((br, LATDIM), lambda i: (i, 0)),
            pl.BlockSpec((br, NREL), lambda i: (i, 0)),
            pl.BlockSpec((br, NREL), lambda i: (i, 0)),
        ],
        out_shape=[
            jax.ShapeDtypeStruct((ENTITY, LATDIM), jnp.float32),
            jax.ShapeDtypeStruct((ENTITY, LATDIM), jnp.float32),
            jax.ShapeDtypeStruct((ENTITY, NREL), jnp.float32),
            jax.ShapeDtypeStruct((ENTITY, NREL), jnp.float32),
        ],
    )(aggp, sp, emb, res, w, r)


def _comb_body(e0_ref, g1_ref, g2_ref, o_ref):
    o_ref[...] = e0_ref[...] + g1_ref[...] + g2_ref[...]


def _combine(e0, g1, g2):
    nb = 16
    br = N_GRAPH // nb
    return pl.pallas_call(
        _comb_body,
        grid=(nb,),
        in_specs=[
            pl.BlockSpec((br, LATDIM), lambda i: (i, 0)),
            pl.BlockSpec((br, LATDIM), lambda i: (i, 0)),
            pl.BlockSpec((br, LATDIM), lambda i: (i, 0)),
        ],
        out_specs=pl.BlockSpec((br, LATDIM), lambda i: (i, 0)),
        out_shape=jax.ShapeDtypeStruct((N_GRAPH, LATDIM), jnp.float32),
    )(e0, g1, g2)


def kernel(adj_rows, adj_cols, adj_vals, edge_index, edge_type,
           uEmbeds, eEmbeds, rEmbeds, W):
    head = edge_index[0].astype(jnp.int32)
    tail = edge_index[1].astype(jnp.int32)
    et = edge_type.astype(jnp.int32)

    npad = E_KG_PAD - E_KG
    ih = jnp.pad(head * NREL + et, (0, npad))
    it = jnp.pad(tail * NREL + et, (0, npad))
    tl = jnp.pad(tail, (0, npad))
    hd = jnp.pad(head, (0, npad), constant_values=DUMP_E)
    cols = adj_cols.astype(jnp.int32)
    rows = adj_rows.astype(jnp.int32)

    zmat_e = jnp.zeros((ENTITY, LATDIM), jnp.float32)
    zcol_e = jnp.zeros((ENTITY,), jnp.float32)

    ah, at = _prep(eEmbeds, W, rEmbeds)
    emb = eEmbeds
    res = eEmbeds
    for _ in range(N_HOPS):
        aggp, sp = _rgat(emb, ah.reshape(-1), at.reshape(-1),
                         ih, it, tl, hd, zmat_e, zcol_e)
        emb, res, ah, at = _hop_update(aggp, sp.reshape(NC, ENTITY).T,
                                       emb, res, W, rEmbeds)

    embeds0 = jnp.concatenate([uEmbeds, res[:ITEM]], axis=0)
    g1 = _spmm(embeds0, cols, rows, adj_vals,
               zmat_e).reshape(N_GRAPH, LATDIM)
    g2 = _spmm(g1, cols, rows, adj_vals, zmat_e).reshape(N_GRAPH, LATDIM)
    total = _combine(embeds0, g1, g2)
    return total[:USER], total[USER:]


# speedup vs baseline: 1.0699x; 1.0152x over previous
import functools

import jax
import jax.numpy as jnp
from jax import lax
from jax.experimental import pallas as pl
from jax.experimental.pallas import tpu as pltpu
from jax.experimental.pallas import tpu_sc as plsc

USER = 10000
ITEM = 6000
ENTITY = 10000
LATDIM = 128
NREL = 16
N_HOPS = 2
GNN_LAYER = 2
RES_LAMBDA = 0.5
NNZ_ADJ = 512000
E_KG = 320000
N_GRAPH = USER + ITEM

NC = 2
NS = 16
NW = NC * NS
L = 16

CH = 128
EPT_KG = 10112
E_KG_PAD = EPT_KG * NW
DUMP_E = ENTITY
HALF = N_GRAPH // 2
DUMP = HALF

_SC_MESH = plsc.VectorSubcoreMesh(
    core_axis_name="c", subcore_axis_name="s", num_cores=NC, num_subcores=NS)


def _rgat_body(emb, ahf, atf, ih, it, tl, hd, zmat, zcol,
               agg_out, s_out,
               acc, sacc, ihb, itb, tlb, hdb, hds, ahv2, atv2, exv2, rows2,
               sv, isem, gsem, ssem):
    cid = lax.axis_index("c")
    sid = lax.axis_index("s")
    wid = sid * NC + cid
    nch = EPT_KG // CH
    rpt = 1000
    c0 = wid * nch

    @pl.when(sid < ENTITY // rpt)
    def _():
        pltpu.sync_copy(zmat.at[pl.ds(sid * rpt, rpt), :],
                        acc.at[pl.ds(sid * rpt, rpt), :])
        pltpu.sync_copy(zcol.at[pl.ds(sid * rpt, rpt)], sv)
        pltpu.sync_copy(sv, sacc.at[pl.ds(sid * rpt, rpt)])

    plsc.subcore_barrier()

    def fire_idx(j, p):
        b = (c0 + j) * CH
        pltpu.async_copy(ih.at[pl.ds(b, CH)], ihb.at[p], isem.at[p])
        pltpu.async_copy(it.at[pl.ds(b, CH)], itb.at[p], isem.at[p])
        pltpu.async_copy(tl.at[pl.ds(b, CH)], tlb.at[p], isem.at[p])
        pltpu.async_copy(hd.at[pl.ds(b, CH)], hdb.at[p], isem.at[p])

    def wait_idx(j, p):
        b = (c0 + j) * CH
        pltpu.make_async_copy(ih.at[pl.ds(b, CH)], ihb.at[p],
                              isem.at[p]).wait()
        pltpu.make_async_copy(it.at[pl.ds(b, CH)], itb.at[p],
                              isem.at[p]).wait()
        pltpu.make_async_copy(tl.at[pl.ds(b, CH)], tlb.at[p],
                              isem.at[p]).wait()
        pltpu.make_async_copy(hd.at[pl.ds(b, CH)], hdb.at[p],
                              isem.at[p]).wait()

    def fire_data(p):
        pltpu.async_copy(ahf.at[ihb.at[p]], ahv2.at[p], gsem.at[p])
        pltpu.async_copy(atf.at[itb.at[p]], atv2.at[p], gsem.at[p])
        pltpu.async_copy(emb.at[tlb.at[p]], rows2.at[p], gsem.at[p])

    def wait_data(p):
        pltpu.make_async_copy(ahf.at[ihb.at[p]], ahv2.at[p],
                              gsem.at[p]).wait()
        pltpu.make_async_copy(atf.at[itb.at[p]], atv2.at[p],
                              gsem.at[p]).wait()
        pltpu.make_async_copy(emb.at[tlb.at[p]], rows2.at[p],
                              gsem.at[p]).wait()

    def fire_scatter(p):
        pltpu.async_copy(exv2.at[p], sacc.at[hds.at[p]], ssem.at[p],
                         add=True)
        pltpu.async_copy(rows2.at[p], acc.at[hds.at[p]], ssem.at[p],
                         add=True)

    def drain_scatter(p):
        pltpu.make_async_copy(exv2.at[p], sacc.at[hds.at[p]],
                              ssem.at[p]).wait()
        pltpu.make_async_copy(rows2.at[p], acc.at[hds.at[p]],
                              ssem.at[p]).wait()

    fire_idx(0, 0)
    fire_idx(1, 1)
    wait_idx(0, 0)
    fire_data(0)

    def body(j, _):
        p = lax.rem(j, 2)
        pn = lax.rem(j + 1, 2)
        wait_data(p)

        @pl.when(j >= 1)
        def _():
            drain_scatter(pn)

        @pl.when(j + 1 < nch)
        def _():
            wait_idx(j + 1, pn)
            fire_data(pn)

        for g in range(CH // L):
            e = ahv2[p, pl.ds(g * L, L)] + atv2[p, pl.ds(g * L, L)]
            e = jnp.where(e >= 0.0, e, 0.2 * e)
            exv2[p, pl.ds(g * L, L)] = jnp.exp(e)
            hds[p, pl.ds(g * L, L)] = hdb[p, pl.ds(g * L, L)]

        for g in range(CH // L):
            scv = exv2[p, pl.ds(g * L, L)]
            for k in range(L):
                sc = scv[k]
                i = g * L + k
                for u in range(LATDIM // L):
                    rows2[p, i, pl.ds(u * L, L)] = (
                        rows2[p, i, pl.ds(u * L, L)] * sc)

        fire_scatter(p)

        @pl.when(j + 2 < nch)
        def _():
            fire_idx(j + 2, p)

        return 0

    lax.fori_loop(0, nch, body, 0)
    drain_scatter(lax.rem(nch - 1, 2))
    plsc.subcore_barrier()

    @pl.when(sid < ENTITY // rpt)
    def _():
        pltpu.sync_copy(acc.at[pl.ds(sid * rpt, rpt), :],
                        agg_out.at[cid, pl.ds(sid * rpt, rpt), :])
        pltpu.sync_copy(sacc.at[pl.ds(sid * rpt, rpt)], sv)
        pltpu.sync_copy(sv, s_out.at[pl.ds(cid * ENTITY + sid * rpt, rpt)])


def _rgat(emb, ahf, atf, ih, it, tl, hd, zmat, zcol):
    f = pl.kernel(
        _rgat_body,
        out_type=(jax.ShapeDtypeStruct((NC, ENTITY, LATDIM), jnp.float32),
                  jax.ShapeDtypeStruct((NC * ENTITY,), jnp.float32)),
        mesh=_SC_MESH,
        scratch_types=[
            pltpu.VMEM_SHARED((ENTITY + 8, LATDIM), jnp.float32),
            pltpu.VMEM_SHARED((ENTITY + 8,), jnp.float32),
            pltpu.VMEM((2, CH), jnp.int32),
            pltpu.VMEM((2, CH), jnp.int32),
            pltpu.VMEM((2, CH), jnp.int32),
            pltpu.VMEM((2, CH), jnp.int32),
            pltpu.VMEM((2, CH), jnp.int32),
            pltpu.VMEM((2, CH), jnp.float32),
            pltpu.VMEM((2, CH), jnp.float32),
            pltpu.VMEM((2, CH), jnp.float32),
            pltpu.VMEM((2, CH, LATDIM), jnp.float32),
            pltpu.VMEM((1000,), jnp.float32),
            pltpu.SemaphoreType.DMA((2,)),
            pltpu.SemaphoreType.DMA((2,)),
            pltpu.SemaphoreType.DMA((2,)),
        ],
    )
    return f(emb, ahf, atf, ih, it, tl, hd, zmat, zcol)


def _spmm_body(prev, cols, rows, vals, zmat,
               part_out,
               acc, colb, rowb, valb, rowm, rows2, isem, gsem, ssem):
    cid = lax.axis_index("c")
    sid = lax.axis_index("s")
    ept = NNZ_ADJ // NS
    nch = ept // CH
    rpt = 1000
    c0 = sid * nch
    lo = cid * HALF

    @pl.when(sid < HALF // rpt)
    def _():
        pltpu.sync_copy(zmat.at[pl.ds(sid * rpt, rpt), :],
                        acc.at[pl.ds(sid * rpt, rpt), :])

    plsc.subcore_barrier()

    def fire_idx(j, p):
        b = (c0 + j) * CH
        pltpu.async_copy(cols.at[pl.ds(b, CH)], colb.at[p], isem.at[p])
        pltpu.async_copy(rows.at[pl.ds(b, CH)], rowb.at[p], isem.at[p])
        pltpu.async_copy(vals.at[pl.ds(b, CH)], valb.at[p], isem.at[p])

    def wait_idx(j, p):
        b = (c0 + j) * CH
        pltpu.make_async_copy(cols.at[pl.ds(b, CH)], colb.at[p],
                              isem.at[p]).wait()
        pltpu.make_async_copy(rows.at[pl.ds(b, CH)], rowb.at[p],
                              isem.at[p]).wait()
        pltpu.make_async_copy(vals.at[pl.ds(b, CH)], valb.at[p],
                              isem.at[p]).wait()

    def fire_data(p):
        pltpu.async_copy(prev.at[colb.at[p]], rows2.at[p], gsem.at[p])

    def wait_data(p):
        pltpu.make_async_copy(prev.at[colb.at[p]], rows2.at[p],
                              gsem.at[p]).wait()

    def fire_scatter(p):
        pltpu.async_copy(rows2.at[p], acc.at[rowm.at[p]], ssem.at[p],
                         add=True)

    def drain_scatter(p):
        pltpu.make_async_copy(rows2.at[p], acc.at[rowm.at[p]],
                              ssem.at[p]).wait()

    fire_idx(0, 0)
    fire_idx(1, 1)
    fire_idx(2, 2)
    wait_idx(0, 0)
    fire_data(0)

    def body(j, _):
        p = lax.rem(j, 3)
        pn = lax.rem(j + 1, 3)
        wait_data(p)

        @pl.when(j >= 2)
        def _():
            drain_scatter(pn)

        @pl.when(j + 1 < nch)
        def _():
            wait_idx(j + 1, pn)
            fire_data(pn)

        for g in range(CH // L):
            r = rowb[p, pl.ds(g * L, L)]
            rl = r - lo
            inr = (rl >= 0) & (rl < HALF)
            rowm[p, pl.ds(g * L, L)] = jnp.where(inr, rl, DUMP)

        for g in range(CH // L):
            scv = valb[p, pl.ds(g * L, L)]
            for k in range(L):
                sc = scv[k]
                i = g * L + k
                for u in range(LATDIM // L):
                    rows2[p, i, pl.ds(u * L, L)] = (
                        rows2[p, i, pl.ds(u * L, L)] * sc)

        fire_scatter(p)

        @pl.when(j + 3 < nch)
        def _():
            fire_idx(j + 3, p)

        return 0

    lax.fori_loop(0, nch, body, 0)
    drain_scatter(lax.rem(nch - 2, 3))
    drain_scatter(lax.rem(nch - 1, 3))
    plsc.subcore_barrier()

    @pl.when(sid < HALF // rpt)
    def _():
        pltpu.sync_copy(acc.at[pl.ds(sid * rpt, rpt), :],
                        part_out.at[cid, pl.ds(sid * rpt, rpt), :])


def _spmm(prev, cols, rows, vals, zmat):
    f = pl.kernel(
        _spmm_body,
        out_type=jax.ShapeDtypeStruct((NC, HALF, LATDIM), jnp.float32),
        mesh=_SC_MESH,
        scratch_types=[
            pltpu.VMEM_SHARED((HALF + 8, LATDIM), jnp.float32),
            pltpu.VMEM((3, CH), jnp.int32),
            pltpu.VMEM((3, CH), jnp.int32),
            pltpu.VMEM((3, CH), jnp.float32),
            pltpu.VMEM((3, CH), jnp.int32),
            pltpu.VMEM((3, CH, LATDIM), jnp.float32),
            pltpu.SemaphoreType.DMA((3,)),
            pltpu.SemaphoreType.DMA((3,)),
            pltpu.SemaphoreType.DMA((3,)),
        ],
    )
    return f(prev, cols, rows, vals, zmat)


def _score_tables(e, w_ref, r_ref):
    wr1 = lax.dot_general(w_ref[:LATDIM, :], r_ref[...],
                          (((1,), (1,)), ((), ())),
                          preferred_element_type=jnp.float32)
    wr2 = lax.dot_general(w_ref[LATDIM:, :], r_ref[...],
                          (((1,), (1,)), ((), ())),
                          preferred_element_type=jnp.float32)
    ah = jnp.dot(e, wr1, preferred_element_type=jnp.float32)
    at = jnp.dot(e, wr2, preferred_element_type=jnp.float32)
    return ah, at


def _prep_body(emb_ref, w_ref, r_ref, ah_ref, at_ref):
    ah, at = _score_tables(emb_ref[...], w_ref, r_ref)
    ah_ref[...] = ah
    at_ref[...] = at


def _prep(emb, w, r):
    nb = 10
    br = ENTITY // nb
    return pl.pallas_call(
        _prep_body,
        grid=(nb,),
        in_specs=[
            pl.BlockSpec((br, LATDIM), lambda i: (i, 0)),
            pl.BlockSpec((2 * LATDIM, LATDIM), lambda i: (0, 0)),
            pl.BlockSpec((NREL, LATDIM), lambda i: (0, 0)),
        ],
        out_specs=[
            pl.BlockSpec((br, NREL), lambda i: (i, 0)),
            pl.BlockSpec((br, NREL), lambda i: (i, 0)),
        ],
        out_shape=[
            jax.ShapeDtypeStruct((ENTITY, NREL), jnp.float32),
            jax.ShapeDtypeStruct((ENTITY, NREL), jnp.float32),
        ],
    )(emb, w, r)


def _hop_body(aggp_ref, sp_ref, emb_ref, res_ref, w_ref, r_ref,
              embo_ref, reso_ref, ah_ref, at_ref):
    agg = aggp_ref[0] + aggp_ref[1]
    s2 = sp_ref[...]
    s = s2[:, 0:1] + s2[:, 1:2]
    denom = jnp.where(s == 0.0, 1.0, s)
    x = agg / denom + emb_ref[...]
    n = jnp.sqrt(jnp.sum(x * x, axis=-1, keepdims=True))
    e = x / jnp.maximum(n, 1e-12)
    embo_ref[...] = e
    reso_ref[...] = RES_LAMBDA * res_ref[...] + e
    ah, at = _score_tables(e, w_ref, r_ref)
    ah_ref[...] = ah
    at_ref[...] = at


def _hop_update(aggp, sp, emb, res, w, r):
    nb = 10
    br = ENTITY // nb
    return pl.pallas_call(
        _hop_body,
        grid=(nb,),
        in_specs=[
            pl.BlockSpec((NC, br, LATDIM), lambda i: (0, i, 0)),
            pl.BlockSpec((br, NC), lambda i: (i, 0)),
            pl.BlockSpec((br, LATDIM), lambda i: (i, 0)),
            pl.BlockSpec((br, LATDIM), lambda i: (i, 0)),
            pl.BlockSpec((2 * LATDIM, LATDIM), lambda i: (0, 0)),
            pl.BlockSpec((NREL, LATDIM), lambda i: (0, 0)),
        ],
        out_specs=[
            pl.BlockSpec((br, LATDIM), lambda i: (i, 0)),
            pl.BlockSpec((br, LATDIM), lambda i: (i, 0)),
            pl.BlockSpec((br, NREL), lambda i: (i, 0)),
            pl.BlockSpec((br, NREL), lambda i: (i, 0)),
        ],
        out_shape=[
            jax.ShapeDtypeStruct((ENTITY, LATDIM), jnp.float32),
            jax.ShapeDtypeStruct((ENTITY, LATDIM), jnp.float32),
            jax.ShapeDtypeStruct((ENTITY, NREL), jnp.float32),
            jax.ShapeDtypeStruct((ENTITY, NREL), jnp.float32),
        ],
    )(aggp, sp, emb, res, w, r)


def _comb_body(e0_ref, g1_ref, g2_ref, o_ref):
    o_ref[...] = e0_ref[...] + g1_ref[...] + g2_ref[...]


def _combine(e0, g1, g2):
    nb = 16
    br = N_GRAPH // nb
    return pl.pallas_call(
        _comb_body,
        grid=(nb,),
        in_specs=[
            pl.BlockSpec((br, LATDIM), lambda i: (i, 0)),
            pl.BlockSpec((br, LATDIM), lambda i: (i, 0)),
            pl.BlockSpec((br, LATDIM), lambda i: (i, 0)),
        ],
        out_specs=pl.BlockSpec((br, LATDIM), lambda i: (i, 0)),
        out_shape=jax.ShapeDtypeStruct((N_GRAPH, LATDIM), jnp.float32),
    )(e0, g1, g2)


def kernel(adj_rows, adj_cols, adj_vals, edge_index, edge_type,
           uEmbeds, eEmbeds, rEmbeds, W):
    head = edge_index[0].astype(jnp.int32)
    tail = edge_index[1].astype(jnp.int32)
    et = edge_type.astype(jnp.int32)

    npad = E_KG_PAD - E_KG
    ih = jnp.pad(head * NREL + et, (0, npad))
    it = jnp.pad(tail * NREL + et, (0, npad))
    tl = jnp.pad(tail, (0, npad))
    hd = jnp.pad(head, (0, npad), constant_values=DUMP_E)
    cols = adj_cols.astype(jnp.int32)
    rows = adj_rows.astype(jnp.int32)

    zmat_e = jnp.zeros((ENTITY, LATDIM), jnp.float32)
    zcol_e = jnp.zeros((ENTITY,), jnp.float32)

    ah, at = _prep(eEmbeds, W, rEmbeds)
    emb = eEmbeds
    res = eEmbeds
    for _ in range(N_HOPS):
        aggp, sp = _rgat(emb, ah.reshape(-1), at.reshape(-1),
                         ih, it, tl, hd, zmat_e, zcol_e)
        emb, res, ah, at = _hop_update(aggp, sp.reshape(NC, ENTITY).T,
                                       emb, res, W, rEmbeds)

    embeds0 = jnp.concatenate([uEmbeds, res[:ITEM]], axis=0)
    g1 = _spmm(embeds0, cols, rows, adj_vals,
               zmat_e).reshape(N_GRAPH, LATDIM)
    g2 = _spmm(g1, cols, rows, adj_vals, zmat_e).reshape(N_GRAPH, LATDIM)
    total = _combine(embeds0, g1, g2)
    return total[:USER], total[USER:]
